# Initial kernel scaffold; baseline (speedup 1.0000x reference)
import jax
import jax.numpy as jnp
from jax.experimental import pallas as pl


def kernel(sent1_enc, sent2_enc, relation_idx, w_t, b, hyperplane_w, relation_embedding):
    raise NotImplementedError("write your pallas kernel here")


# speedup vs baseline: 1.1398x; 1.1398x over previous
import jax
import jax.numpy as jnp
from jax.experimental import pallas as pl
from jax.experimental.pallas import tpu as pltpu


def _transh_kernel(s1_ref, s2_ref, idx_ref, w_ref, b_ref, hw_ref, re_ref,
                   out1_ref, out2_ref, rel_ref, wr_ref):
    idx = idx_ref[...]
    bt = idx.shape[0]
    r = hw_ref.shape[0]

    iota_r = jax.lax.broadcasted_iota(jnp.int32, (bt, r), 1)
    one_hot = (idx == iota_r).astype(jnp.bfloat16)
    w_r = jnp.dot(one_hot, hw_ref[...], preferred_element_type=jnp.float32)
    rel = jnp.dot(one_hot, re_ref[...], preferred_element_type=jnp.float32)

    w = w_ref[...]
    b = b_ref[...]
    h1 = jnp.dot(s1_ref[...].astype(jnp.bfloat16), w,
                 preferred_element_type=jnp.float32) + b
    h2 = jnp.dot(s2_ref[...].astype(jnp.bfloat16), w,
                 preferred_element_type=jnp.float32) + b

    out1_ref[...] = h1 - jnp.sum(w_r * h1, axis=-1, keepdims=True) * w_r
    out2_ref[...] = h2 - jnp.sum(w_r * h2, axis=-1, keepdims=True) * w_r
    rel_ref[...] = rel
    wr_ref[...] = w_r


def kernel(sent1_enc, sent2_enc, relation_idx, w_t, b,
           hyperplane_w, relation_embedding):
    B, S = sent1_enc.shape
    M = w_t.shape[1]
    R = hyperplane_w.shape[0]

    idx2d = relation_idx.reshape(B, 1).astype(jnp.int32)
    b2d = b.reshape(1, M)
    w_bf = w_t.astype(jnp.bfloat16)
    hw_bf = hyperplane_w.astype(jnp.bfloat16)
    re_bf = relation_embedding.astype(jnp.bfloat16)

    bt = min(512, B)
    grid = (pl.cdiv(B, bt),)
    out_shapes = tuple(jax.ShapeDtypeStruct((B, M), jnp.float32)
                       for _ in range(4))
    return pl.pallas_call(
        _transh_kernel,
        out_shape=out_shapes,
        grid=grid,
        in_specs=[
            pl.BlockSpec((bt, S), lambda i: (i, 0)),
            pl.BlockSpec((bt, S), lambda i: (i, 0)),
            pl.BlockSpec((bt, 1), lambda i: (i, 0)),
            pl.BlockSpec((S, M), lambda i: (0, 0)),
            pl.BlockSpec((1, M), lambda i: (0, 0)),
            pl.BlockSpec((R, M), lambda i: (0, 0)),
            pl.BlockSpec((R, M), lambda i: (0, 0)),
        ],
        out_specs=(
            pl.BlockSpec((bt, M), lambda i: (i, 0)),
            pl.BlockSpec((bt, M), lambda i: (i, 0)),
            pl.BlockSpec((bt, M), lambda i: (i, 0)),
            pl.BlockSpec((bt, M), lambda i: (i, 0)),
        ),
        compiler_params=pltpu.CompilerParams(
            dimension_semantics=("parallel",)),
    )(sent1_enc, sent2_enc, idx2d, w_bf, b2d, hw_bf, re_bf)


# speedup vs baseline: 1.2605x; 1.1059x over previous
import jax
import jax.numpy as jnp
from jax.experimental import pallas as pl
from jax.experimental.pallas import tpu as pltpu


def _transh_kernel(s1_ref, s2_ref, idx_ref, w_ref, b_ref, hw_ref, re_ref,
                   out1_ref, out2_ref, rel_ref, wr_ref):
    idx = idx_ref[...]
    bt = idx.shape[0]
    r = hw_ref.shape[0]

    iota_r = jax.lax.broadcasted_iota(jnp.int32, (bt, r), 1)
    one_hot = (idx == iota_r).astype(jnp.bfloat16)
    w_r = jnp.dot(one_hot, hw_ref[...], preferred_element_type=jnp.float32)
    rel = jnp.dot(one_hot, re_ref[...], preferred_element_type=jnp.float32)

    w = w_ref[...]
    b = b_ref[...]
    h1 = jnp.dot(s1_ref[...].astype(jnp.bfloat16), w,
                 preferred_element_type=jnp.float32) + b
    h2 = jnp.dot(s2_ref[...].astype(jnp.bfloat16), w,
                 preferred_element_type=jnp.float32) + b

    out1_ref[...] = h1 - jnp.sum(w_r * h1, axis=-1, keepdims=True) * w_r
    out2_ref[...] = h2 - jnp.sum(w_r * h2, axis=-1, keepdims=True) * w_r
    rel_ref[...] = rel
    wr_ref[...] = w_r


def kernel(sent1_enc, sent2_enc, relation_idx, w_t, b,
           hyperplane_w, relation_embedding):
    B, S = sent1_enc.shape
    M = w_t.shape[1]
    R = hyperplane_w.shape[0]

    idx2d = relation_idx.reshape(B, 1).astype(jnp.int32)
    b2d = b.reshape(1, M)
    w_bf = w_t.astype(jnp.bfloat16)
    hw_bf = hyperplane_w.astype(jnp.bfloat16)
    re_bf = relation_embedding.astype(jnp.bfloat16)

    bt = min(1024, B)
    grid = (pl.cdiv(B, bt),)
    out_shapes = tuple(jax.ShapeDtypeStruct((B, M), jnp.float32)
                       for _ in range(4))
    return pl.pallas_call(
        _transh_kernel,
        out_shape=out_shapes,
        grid=grid,
        in_specs=[
            pl.BlockSpec((bt, S), lambda i: (i, 0)),
            pl.BlockSpec((bt, S), lambda i: (i, 0)),
            pl.BlockSpec((bt, 1), lambda i: (i, 0)),
            pl.BlockSpec((S, M), lambda i: (0, 0)),
            pl.BlockSpec((1, M), lambda i: (0, 0)),
            pl.BlockSpec((R, M), lambda i: (0, 0)),
            pl.BlockSpec((R, M), lambda i: (0, 0)),
        ],
        out_specs=(
            pl.BlockSpec((bt, M), lambda i: (i, 0)),
            pl.BlockSpec((bt, M), lambda i: (i, 0)),
            pl.BlockSpec((bt, M), lambda i: (i, 0)),
            pl.BlockSpec((bt, M), lambda i: (i, 0)),
        ),
        compiler_params=pltpu.CompilerParams(
            dimension_semantics=("parallel",)),
    )(sent1_enc, sent2_enc, idx2d, w_bf, b2d, hw_bf, re_bf)


# speedup vs baseline: 1.2898x; 1.0232x over previous
import jax
import jax.numpy as jnp
from jax.experimental import pallas as pl
from jax.experimental.pallas import tpu as pltpu


def _transh_kernel(s1_ref, s2_ref, idx_ref, w_ref, b_ref, tbl_ref,
                   out1_ref, out2_ref, rel_ref, wr_ref):
    idx = idx_ref[...]
    bt = idx.shape[0]
    r = tbl_ref.shape[0]
    m = out1_ref.shape[1]

    iota_r = jax.lax.broadcasted_iota(jnp.int32, (bt, r), 1)
    one_hot = (idx == iota_r).astype(jnp.bfloat16)
    wr_rel = jnp.dot(one_hot, tbl_ref[...], preferred_element_type=jnp.float32)
    w_r = wr_rel[:, :m]

    w = w_ref[...]
    b = b_ref[...]
    h1 = jnp.dot(s1_ref[...].astype(jnp.bfloat16), w,
                 preferred_element_type=jnp.float32) + b
    h2 = jnp.dot(s2_ref[...].astype(jnp.bfloat16), w,
                 preferred_element_type=jnp.float32) + b

    out1_ref[...] = h1 - jnp.sum(w_r * h1, axis=-1, keepdims=True) * w_r
    out2_ref[...] = h2 - jnp.sum(w_r * h2, axis=-1, keepdims=True) * w_r
    rel_ref[...] = wr_rel[:, m:]
    wr_ref[...] = w_r


def kernel(sent1_enc, sent2_enc, relation_idx, w_t, b,
           hyperplane_w, relation_embedding):
    B, S = sent1_enc.shape
    M = w_t.shape[1]
    R = hyperplane_w.shape[0]

    idx2d = relation_idx.reshape(B, 1).astype(jnp.int32)
    b2d = b.reshape(1, M)
    w_bf = w_t.astype(jnp.bfloat16)
    tbl_bf = jnp.concatenate(
        [hyperplane_w.astype(jnp.bfloat16),
         relation_embedding.astype(jnp.bfloat16)], axis=1)

    bt = min(1024, B)
    grid = (pl.cdiv(B, bt),)
    out_shapes = tuple(jax.ShapeDtypeStruct((B, M), jnp.float32)
                       for _ in range(4))
    return pl.pallas_call(
        _transh_kernel,
        out_shape=out_shapes,
        grid=grid,
        in_specs=[
            pl.BlockSpec((bt, S), lambda i: (i, 0)),
            pl.BlockSpec((bt, S), lambda i: (i, 0)),
            pl.BlockSpec((bt, 1), lambda i: (i, 0)),
            pl.BlockSpec((S, M), lambda i: (0, 0)),
            pl.BlockSpec((1, M), lambda i: (0, 0)),
            pl.BlockSpec((R, 2 * M), lambda i: (0, 0)),
        ],
        out_specs=(
            pl.BlockSpec((bt, M), lambda i: (i, 0)),
            pl.BlockSpec((bt, M), lambda i: (i, 0)),
            pl.BlockSpec((bt, M), lambda i: (i, 0)),
            pl.BlockSpec((bt, M), lambda i: (i, 0)),
        ),
        compiler_params=pltpu.CompilerParams(
            dimension_semantics=("parallel",)),
    )(sent1_enc, sent2_enc, idx2d, w_bf, b2d, tbl_bf)


# speedup vs baseline: 1.3584x; 1.0532x over previous
import jax
import jax.numpy as jnp
from jax.experimental import pallas as pl
from jax.experimental.pallas import tpu as pltpu


def _transh_kernel(s1_ref, s2_ref, idx_ref, w_ref, b_ref, hw_ref, re_ref,
                   out1_ref, out2_ref, rel_ref, wr_ref):
    idx = idx_ref[...]
    bt = idx.shape[0]
    r = hw_ref.shape[0]

    iota_r = jax.lax.broadcasted_iota(jnp.int32, (bt, r), 1)
    one_hot = (idx == iota_r).astype(jnp.bfloat16)
    w_r = jnp.dot(one_hot, hw_ref[...].astype(jnp.bfloat16),
                  preferred_element_type=jnp.float32)
    rel = jnp.dot(one_hot, re_ref[...].astype(jnp.bfloat16),
                  preferred_element_type=jnp.float32)

    w = w_ref[...].astype(jnp.bfloat16)
    b = b_ref[...]
    h1 = jnp.dot(s1_ref[...].astype(jnp.bfloat16), w,
                 preferred_element_type=jnp.float32) + b
    h2 = jnp.dot(s2_ref[...].astype(jnp.bfloat16), w,
                 preferred_element_type=jnp.float32) + b

    out1_ref[...] = h1 - jnp.sum(w_r * h1, axis=-1, keepdims=True) * w_r
    out2_ref[...] = h2 - jnp.sum(w_r * h2, axis=-1, keepdims=True) * w_r
    rel_ref[...] = rel
    wr_ref[...] = w_r


def kernel(sent1_enc, sent2_enc, relation_idx, w_t, b,
           hyperplane_w, relation_embedding):
    B, S = sent1_enc.shape
    M = w_t.shape[1]
    R = hyperplane_w.shape[0]

    idx2d = relation_idx.reshape(B, 1).astype(jnp.int32)
    b2d = b.reshape(1, M)

    bt = min(1024, B)
    grid = (pl.cdiv(B, bt),)
    out_shapes = tuple(jax.ShapeDtypeStruct((B, M), jnp.float32)
                       for _ in range(4))
    return pl.pallas_call(
        _transh_kernel,
        out_shape=out_shapes,
        grid=grid,
        in_specs=[
            pl.BlockSpec((bt, S), lambda i: (i, 0)),
            pl.BlockSpec((bt, S), lambda i: (i, 0)),
            pl.BlockSpec((bt, 1), lambda i: (i, 0)),
            pl.BlockSpec((S, M), lambda i: (0, 0)),
            pl.BlockSpec((1, M), lambda i: (0, 0)),
            pl.BlockSpec((R, M), lambda i: (0, 0)),
            pl.BlockSpec((R, M), lambda i: (0, 0)),
        ],
        out_specs=(
            pl.BlockSpec((bt, M), lambda i: (i, 0)),
            pl.BlockSpec((bt, M), lambda i: (i, 0)),
            pl.BlockSpec((bt, M), lambda i: (i, 0)),
            pl.BlockSpec((bt, M), lambda i: (i, 0)),
        ),
        compiler_params=pltpu.CompilerParams(
            dimension_semantics=("parallel",)),
    )(sent1_enc, sent2_enc, idx2d, w_t, b2d,
      hyperplane_w, relation_embedding)
